# baseline (speedup 1.0000x reference)
import jax
import jax.numpy as jnp
from jax import lax
from jax.experimental import pallas as pl
from jax.experimental.pallas import tpu as pltpu
from jax.experimental.pallas import tpu_sc as plsc

VOCAB_P1 = 1000001
DIM = 64
B = 16384
L = 20

NC = 2
NS = 16
NW = NC * NS

SEQ_PER_W = B // NW
LANES = 16
VREGS_PER_ROW = DIM // LANES
NBUF = 8


def _body(idx_hbm, table_hbm, out_hbm, idx_v, rowbuf, outbuf, *sems):
    cid = lax.axis_index("c")
    sid = lax.axis_index("s")
    wid = sid * NC + cid

    pltpu.sync_copy(idx_hbm.at[pl.ds(wid * SEQ_PER_W, SEQ_PER_W)], idx_v)

    def fire(s, buf):
        pltpu.async_copy(table_hbm.at[idx_v.at[s]], rowbuf.at[buf], sems[buf])

    def drain(buf):
        pltpu.make_async_copy(
            table_hbm.at[pl.ds(0, L)], rowbuf.at[buf], sems[buf]
        ).wait()

    def compute(s, buf):
        inv_l = jnp.float32(1.0 / L)
        lanes = [pl.ds(q * LANES, LANES) for q in range(VREGS_PER_ROW)]
        accs = [rowbuf[buf, 0, lane] for lane in lanes]
        for l in range(1, L):
            for q in range(VREGS_PER_ROW):
                accs[q] = accs[q] + rowbuf[buf, l, lanes[q]]
        for q in range(VREGS_PER_ROW):
            outbuf[s, lanes[q]] = accs[q] * inv_l

    for p in range(NBUF - 1):
        fire(p, p)

    @pl.loop(0, SEQ_PER_W, step=NBUF)
    def _outer(s0):
        for b in range(NBUF):
            s = s0 + b
            nxt = s + NBUF - 1

            @pl.when(nxt < SEQ_PER_W)
            def _():
                fire(nxt, (b + NBUF - 1) % NBUF)

            drain(b)
            compute(s, b)

    pltpu.sync_copy(outbuf, out_hbm.at[pl.ds(wid * SEQ_PER_W, SEQ_PER_W)])


@jax.jit
def kernel(indices, table):
    f = pl.kernel(
        _body,
        out_type=jax.ShapeDtypeStruct((B, DIM), jnp.float32),
        mesh=plsc.VectorSubcoreMesh(core_axis_name="c", subcore_axis_name="s"),
        scratch_types=[
            pltpu.VMEM((SEQ_PER_W, L), jnp.int32),
            pltpu.VMEM((NBUF, L, DIM), jnp.float32),
            pltpu.VMEM((SEQ_PER_W, DIM), jnp.float32),
        ] + [pltpu.SemaphoreType.DMA] * NBUF,
        compiler_params=pltpu.CompilerParams(use_tc_tiling_on_sc=False),
    )
    return f(indices, table)
